# baseline (speedup 1.0000x reference)
import jax
import jax.numpy as jnp
from jax.experimental import pallas as pl
from jax.experimental.pallas import tpu as pltpu

B, S, D = 4, 8192, 2048
S_BLK = 1024
NS = S // S_BLK
LANES = 128


def _copy_reduce_kernel(idx_ref, x_ref, out_ref, vals_ref, acc_ref, mins_ref):
    j = pl.program_id(0)
    s = pl.program_id(1)
    blk = x_ref[0]
    out_ref[0] = blk
    part = jnp.min(blk, axis=0, keepdims=True)

    @pl.when(s == 0)
    def _():
        acc_ref[...] = part

    @pl.when(s != 0)
    def _():
        acc_ref[...] = jnp.minimum(acc_ref[...], part)

    @pl.when(s == NS - 1)
    def _():
        acc = acc_ref[...]
        idx = idx_ref[j]
        lane = jax.lax.broadcasted_iota(jnp.int32, (1, D), 1)
        mins_ref[j, 0] = jnp.min(acc)
        mins_ref[j, 1] = jnp.min(jnp.where(lane == idx, jnp.inf, acc))

    @pl.when((j == B - 1) & (s == NS - 1))
    def _():
        prefix = jnp.float32(jnp.inf)
        for i in range(B):
            suf = mins_ref[i, 0]
            for k in range(i + 1, B):
                suf = jnp.minimum(suf, mins_ref[k, 0])
            m = jnp.minimum(prefix, suf)
            v = jnp.where(m == 0.0, jnp.float32(0.0), m - jnp.float32(100000.0))
            vals_ref[i] = v
            prefix = jnp.minimum(prefix, jnp.minimum(mins_ref[i, 1], v))


F_BLK = 1024
NF = S // F_BLK


def _fixup_kernel(idx_ref, vals_ref, big_ref, out_ref):
    i = pl.program_id(0)
    v = vals_ref[i]
    col = idx_ref[i] % LANES
    lane = jax.lax.broadcasted_iota(jnp.int32, (1, F_BLK, LANES), 2)
    out_ref[...] = jnp.where(lane == col, v, big_ref[...])


def kernel(x, indices):
    indices = indices.astype(jnp.int32)
    big, vals = pl.pallas_call(
        _copy_reduce_kernel,
        grid_spec=pltpu.PrefetchScalarGridSpec(
            num_scalar_prefetch=1,
            grid=(B, NS),
            in_specs=[
                pl.BlockSpec((1, S_BLK, D), lambda j, s, idx: (j, s, 0)),
            ],
            out_specs=[
                pl.BlockSpec((1, S_BLK, D), lambda j, s, idx: (j, s, 0)),
                pl.BlockSpec(memory_space=pltpu.SMEM),
            ],
            scratch_shapes=[
                pltpu.VMEM((1, D), jnp.float32),
                pltpu.SMEM((B, 2), jnp.float32),
            ],
        ),
        out_shape=[
            jax.ShapeDtypeStruct((B, S, D), jnp.float32),
            jax.ShapeDtypeStruct((B,), jnp.float32),
        ],
    )(indices, x)

    out = pl.pallas_call(
        _fixup_kernel,
        grid_spec=pltpu.PrefetchScalarGridSpec(
            num_scalar_prefetch=1,
            grid=(B, NF),
            in_specs=[
                pl.BlockSpec(memory_space=pltpu.SMEM),
                pl.BlockSpec(
                    (1, F_BLK, LANES), lambda i, f, idx: (i, f, idx[i] // LANES)
                ),
            ],
            out_specs=pl.BlockSpec(
                (1, F_BLK, LANES), lambda i, f, idx: (i, f, idx[i] // LANES)
            ),
        ),
        out_shape=jax.ShapeDtypeStruct((B, S, D), jnp.float32),
        input_output_aliases={2: 0},
    )(indices, vals, big)
    return out


# speedup vs baseline: 1.0723x; 1.0723x over previous
import jax
import jax.numpy as jnp
from jax.experimental import pallas as pl
from jax.experimental.pallas import tpu as pltpu

B, S, D = 4, 8192, 2048
S_BLK = 1024
NS = S // S_BLK
LANES = 128


def _copy_reduce_kernel(idx_ref, x_ref, out_ref, vals_ref, acc_ref, mins_ref):
    j = pl.program_id(0)
    s = pl.program_id(1)
    blk = x_ref[0]
    out_ref[0] = blk
    part = jnp.min(blk, axis=0, keepdims=True)

    @pl.when(s == 0)
    def _():
        acc_ref[...] = part

    @pl.when(s != 0)
    def _():
        acc_ref[...] = jnp.minimum(acc_ref[...], part)

    @pl.when(s == NS - 1)
    def _():
        acc = acc_ref[...]
        idx = idx_ref[j]
        lane = jax.lax.broadcasted_iota(jnp.int32, (1, D), 1)
        mins_ref[j, 0] = jnp.min(acc)
        mins_ref[j, 1] = jnp.min(jnp.where(lane == idx, jnp.inf, acc))

    @pl.when((j == B - 1) & (s == NS - 1))
    def _():
        prefix = jnp.float32(jnp.inf)
        for i in range(B):
            suf = mins_ref[i, 0]
            for k in range(i + 1, B):
                suf = jnp.minimum(suf, mins_ref[k, 0])
            m = jnp.minimum(prefix, suf)
            v = jnp.where(m == 0.0, jnp.float32(0.0), m - jnp.float32(100000.0))
            vals_ref[i] = v
            prefix = jnp.minimum(prefix, jnp.minimum(mins_ref[i, 1], v))


F_BLK = S
NF = S // F_BLK


def _fixup_kernel(idx_ref, vals_ref, big_ref, out_ref):
    i = pl.program_id(0)
    v = vals_ref[i]
    col = idx_ref[i] % LANES
    lane = jax.lax.broadcasted_iota(jnp.int32, (1, F_BLK, LANES), 2)
    out_ref[...] = jnp.where(lane == col, v, big_ref[...])


def kernel(x, indices):
    indices = indices.astype(jnp.int32)
    big, vals = pl.pallas_call(
        _copy_reduce_kernel,
        grid_spec=pltpu.PrefetchScalarGridSpec(
            num_scalar_prefetch=1,
            grid=(B, NS),
            in_specs=[
                pl.BlockSpec((1, S_BLK, D), lambda j, s, idx: (j, s, 0)),
            ],
            out_specs=[
                pl.BlockSpec((1, S_BLK, D), lambda j, s, idx: (j, s, 0)),
                pl.BlockSpec(memory_space=pltpu.SMEM),
            ],
            scratch_shapes=[
                pltpu.VMEM((1, D), jnp.float32),
                pltpu.SMEM((B, 2), jnp.float32),
            ],
        ),
        out_shape=[
            jax.ShapeDtypeStruct((B, S, D), jnp.float32),
            jax.ShapeDtypeStruct((B,), jnp.float32),
        ],
    )(indices, x)

    out = pl.pallas_call(
        _fixup_kernel,
        grid_spec=pltpu.PrefetchScalarGridSpec(
            num_scalar_prefetch=1,
            grid=(B, NF),
            in_specs=[
                pl.BlockSpec(memory_space=pltpu.SMEM),
                pl.BlockSpec(
                    (1, F_BLK, LANES), lambda i, f, idx: (i, f, idx[i] // LANES)
                ),
            ],
            out_specs=pl.BlockSpec(
                (1, F_BLK, LANES), lambda i, f, idx: (i, f, idx[i] // LANES)
            ),
        ),
        out_shape=jax.ShapeDtypeStruct((B, S, D), jnp.float32),
        input_output_aliases={2: 0},
    )(indices, vals, big)
    return out
